# baseline (speedup 1.0000x reference)
import math

import jax
import jax.numpy as jnp
from jax import lax
from jax.experimental import pallas as pl
from jax.experimental.pallas import tpu as pltpu
from jax.experimental.pallas import tpu_sc as plsc

VOCAB = 1000000
D = 64
SCALE = math.sqrt(D)
L = 16

NC = 2
NS = 16
NW = NC * NS

S = 4096
P = 200
SB = S // 128

TCOLS = (VOCAB + 127) // 128
K1_ITERS = (TCOLS + NW - 1) // NW
T2_ROWS = VOCAB // 2

GROUPS = (P * SB) // 2
G_PER_W = GROUPS // NW


def _k1_body(embt_hbm, tail_hbm, t2_hbm, buf_v, tbuf_v, sem_i, sem_o):
    wid = lax.axis_index("s") * NC + lax.axis_index("c")

    iota = lax.iota(jnp.int32, L)
    row_sel = [iota + L * k for k in range(D // L)]

    def col_start(i):
        return pl.multiple_of(jnp.minimum(i * NW + wid, TCOLS - 2) * 128, 128)

    def in_start(i, b):
        pltpu.make_async_copy(
            embt_hbm.at[:, pl.ds(col_start(i), 128)], buf_v.at[b], sem_i.at[b]
        ).start()

    def in_wait(i, b):
        pltpu.make_async_copy(
            embt_hbm.at[:, pl.ds(0, 128)], buf_v.at[b], sem_i.at[b]
        ).wait()

    def out_start(i, b):
        pltpu.make_async_copy(
            tbuf_v.at[b],
            t2_hbm.at[pl.ds(pl.multiple_of(col_start(i) // 2, 64), 64)],
            sem_o.at[b],
        ).start()

    def out_wait(b):
        pltpu.make_async_copy(
            tbuf_v.at[b], t2_hbm.at[pl.ds(0, 64)], sem_o.at[b]
        ).wait()

    in_start(0, 0)

    def step(i, _):
        for b in range(2):
            it = i * 2 + b

            @pl.when(it + 1 < K1_ITERS)
            def _():
                in_start(it + 1, 1 - b)

            @pl.when(it >= 2)
            def _():
                out_wait(b)

            in_wait(it, b)
            buf = buf_v.at[b]
            tbuf = tbuf_v.at[b]

            @plsc.parallel_loop(0, 128, step=1, unroll=8)
            def _(d):
                col = jnp.broadcast_to(d, (L,))
                half = (d & 1) * D
                for k in range(D // L):
                    vals = plsc.load_gather(buf, [row_sel[k], col])
                    tbuf[d >> 1, pl.ds(half + L * k, L)] = vals

            out_start(it, b)
        return 0

    lax.fori_loop(0, K1_ITERS // 2, step, 0)
    it = K1_ITERS - 1

    @pl.when(it >= 2)
    def _():
        out_wait(0)

    in_wait(it, 0)
    buf = buf_v.at[0]
    tbuf = tbuf_v.at[0]

    @plsc.parallel_loop(0, 128, step=1, unroll=8)
    def _(d):
        col = jnp.broadcast_to(d, (L,))
        half = (d & 1) * D
        for k in range(D // L):
            vals = plsc.load_gather(buf, [row_sel[k], col])
            tbuf[d >> 1, pl.ds(half + L * k, L)] = vals

    out_start(it, 0)
    out_wait(1)
    out_wait(0)

    @pl.when(wid == 0)
    def _():
        pltpu.sync_copy(tail_hbm, buf_v.at[0, pl.ds(0, 32)])
        pltpu.sync_copy(buf_v.at[0, pl.ds(0, 32)], t2_hbm.at[pl.ds(T2_ROWS - 32, 32)])


def _k2_body(tok_hbm, table_hbm, out_hbm, idx_v, rows_v, yblk_v, sem_i, sem_g, sem_o):
    wid = lax.axis_index("s") * NC + lax.axis_index("c")

    iota = lax.iota(jnp.int32, L)
    row_sel = [iota + L * j for j in range(256 // L)]

    def group_psb(gl):
        g = gl * NW + wid
        return g // 16, 2 * lax.rem(g, 16)

    def idx_start(gl, b):
        p, sb0 = group_psb(gl)
        pltpu.make_async_copy(
            tok_hbm.at[p, pl.ds(sb0 * 128, 256)], idx_v.at[b], sem_i.at[b]
        ).start()

    def idx_wait(b):
        pltpu.make_async_copy(
            tok_hbm.at[0, pl.ds(0, 256)], idx_v.at[b], sem_i.at[b]
        ).wait()

    def gather_start(b):
        for h in range(2):
            pltpu.make_async_copy(
                table_hbm.at[idx_v.at[b, pl.ds(128 * h, 128)]],
                rows_v.at[b, pl.ds(128 * h, 128)],
                sem_g.at[b],
            ).start()

    def gather_wait(b):
        for h in range(2):
            pltpu.make_async_copy(
                table_hbm.at[idx_v.at[b, pl.ds(0, 128)]],
                rows_v.at[b, pl.ds(128 * h, 128)],
                sem_g.at[b],
            ).wait()

    def out_start(gl, b):
        p, sb0 = group_psb(gl)
        for ss in range(2):
            for a in range(8):
                pltpu.make_async_copy(
                    yblk_v.at[b, ss, pl.ds(8 * a, 8)],
                    out_hbm.at[p, a, sb0 + ss],
                    sem_o.at[b],
                ).start()

    def out_wait(b):
        for _ in range(16):
            pltpu.make_async_copy(
                yblk_v.at[b, 0, pl.ds(0, 8)], out_hbm.at[0, 0, 0], sem_o.at[b]
            ).wait()

    idx_start(0, 0)
    idx_start(1, 1)
    idx_wait(0)
    gather_start(0)

    def step(go, _):
        for b in range(2):
            gl = go * 2 + b

            @pl.when(gl + 1 < G_PER_W)
            def _():
                idx_wait(1 - b)
                gather_start(1 - b)

            @pl.when(gl >= 2)
            def _():
                out_wait(b)

            gather_wait(b)

            @pl.when(gl + 2 < G_PER_W)
            def _():
                idx_start(gl + 2, b)

            rows = rows_v.at[b]

            for ss in range(2):
                yb = yblk_v.at[b, ss]

                @plsc.parallel_loop(0, D, step=1, unroll=8)
                def _(r):
                    col = jnp.broadcast_to(r, (L,))
                    for j in range(8):
                        vals = plsc.load_gather(
                            rows, [row_sel[8 * ss + j], col]
                        )
                        yb[r, pl.ds(L * j, L)] = vals * SCALE

            out_start(gl, b)
        return 0

    lax.fori_loop(0, G_PER_W // 2, step, 0)
    out_wait(0)
    out_wait(1)


def kernel(tokens, embedding):
    tok_t = tokens.astype(jnp.int32).T
    emb_t = embedding.T
    mesh = plsc.VectorSubcoreMesh(core_axis_name="c", subcore_axis_name="s")

    tail = embedding[VOCAB - 64 :, :].reshape(32, 128)
    t2 = pl.kernel(
        _k1_body,
        out_type=jax.ShapeDtypeStruct((T2_ROWS, 128), jnp.float32),
        mesh=mesh,
        scratch_types=[
            pltpu.VMEM((2, D, 128), jnp.float32),
            pltpu.VMEM((2, D, 128), jnp.float32),
            pltpu.SemaphoreType.DMA((2,)),
            pltpu.SemaphoreType.DMA((2,)),
        ],
        compiler_params=pltpu.CompilerParams(
            use_tc_tiling_on_sc=True, needs_layout_passes=False
        ),
    )(emb_t, tail)

    table = t2.reshape(2 * T2_ROWS, D)

    out5 = pl.kernel(
        _k2_body,
        out_type=jax.ShapeDtypeStruct((P, 8, SB, 8, 128), jnp.float32),
        mesh=mesh,
        scratch_types=[
            pltpu.VMEM((2, 256), jnp.int32),
            pltpu.VMEM((2, 256, D), jnp.float32),
            pltpu.VMEM((2, 2, D, 128), jnp.float32),
            pltpu.SemaphoreType.DMA((2,)),
            pltpu.SemaphoreType.DMA((2,)),
            pltpu.SemaphoreType.DMA((2,)),
        ],
        compiler_params=pltpu.CompilerParams(
            use_tc_tiling_on_sc=False, needs_layout_passes=False
        ),
    )(tok_t, table)

    return out5.transpose(2, 4, 0, 1, 3).reshape(S, P, D)


# speedup vs baseline: 1.0626x; 1.0626x over previous
import math

import jax
import jax.numpy as jnp
from jax import lax
from jax.experimental import pallas as pl
from jax.experimental.pallas import tpu as pltpu
from jax.experimental.pallas import tpu_sc as plsc

VOCAB = 1000000
D = 64
SCALE = math.sqrt(D)
L = 16

NC = 2
NS = 16
NW = NC * NS

S = 4096
P = 200
SB = S // 128

TCOLS = (VOCAB + 127) // 128
K1_ITERS = (TCOLS + NW - 1) // NW
T2_ROWS = VOCAB // 2

GROUPS = (P * SB) // 2
G_PER_W = GROUPS // NW


def _k1_body(embt_hbm, tail_hbm, t2_hbm, buf_v, tbuf_v, sem_i, sem_o):
    wid = lax.axis_index("s") * NC + lax.axis_index("c")

    iota = lax.iota(jnp.int32, L)
    row_sel = [iota + L * k for k in range(D // L)]

    def col_start(i):
        return pl.multiple_of(jnp.minimum(i * NW + wid, TCOLS - 2) * 128, 128)

    def in_start(i, b):
        pltpu.make_async_copy(
            embt_hbm.at[:, pl.ds(col_start(i), 128)], buf_v.at[b], sem_i.at[b]
        ).start()

    def in_wait(i, b):
        pltpu.make_async_copy(
            embt_hbm.at[:, pl.ds(0, 128)], buf_v.at[b], sem_i.at[b]
        ).wait()

    def out_start(i, b):
        pltpu.make_async_copy(
            tbuf_v.at[b],
            t2_hbm.at[pl.ds(pl.multiple_of(col_start(i) // 2, 64), 64)],
            sem_o.at[b],
        ).start()

    def out_wait(b):
        pltpu.make_async_copy(
            tbuf_v.at[b], t2_hbm.at[pl.ds(0, 64)], sem_o.at[b]
        ).wait()

    in_start(0, 0)

    def step(i, _):
        for b in range(2):
            it = i * 2 + b

            @pl.when(it + 1 < K1_ITERS)
            def _():
                in_start(it + 1, 1 - b)

            @pl.when(it >= 2)
            def _():
                out_wait(b)

            in_wait(it, b)
            buf = buf_v.at[b]
            tbuf = tbuf_v.at[b]

            @plsc.parallel_loop(0, 64, step=1, unroll=4)
            def _(q):
                trow = tbuf.at[q]
                for par in range(2):
                    col = jnp.broadcast_to(2 * q + par, (L,))
                    for k in range(D // L):
                        vals = plsc.load_gather(buf, [row_sel[k], col])
                        trow[pl.ds(par * D + L * k, L)] = vals

            out_start(it, b)
        return 0

    lax.fori_loop(0, K1_ITERS // 2, step, 0)
    it = K1_ITERS - 1

    @pl.when(it >= 2)
    def _():
        out_wait(0)

    in_wait(it, 0)
    buf = buf_v.at[0]
    tbuf = tbuf_v.at[0]

    @plsc.parallel_loop(0, 64, step=1, unroll=4)
    def _(q):
        trow = tbuf.at[q]
        for par in range(2):
            col = jnp.broadcast_to(2 * q + par, (L,))
            for k in range(D // L):
                vals = plsc.load_gather(buf, [row_sel[k], col])
                trow[pl.ds(par * D + L * k, L)] = vals

    out_start(it, 0)
    out_wait(1)
    out_wait(0)

    @pl.when(wid == 0)
    def _():
        pltpu.sync_copy(tail_hbm, buf_v.at[0, pl.ds(0, 32)])
        pltpu.sync_copy(buf_v.at[0, pl.ds(0, 32)], t2_hbm.at[pl.ds(T2_ROWS - 32, 32)])


def _k2_body(tok_hbm, table_hbm, out_hbm, idx_v, rows_v, yblk_v, sem_i, sem_g, sem_o):
    wid = lax.axis_index("s") * NC + lax.axis_index("c")

    iota = lax.iota(jnp.int32, L)
    row_sel = [iota + L * j for j in range(256 // L)]

    def group_psb(gl):
        g = gl * NW + wid
        return g // 16, 2 * lax.rem(g, 16)

    def idx_start(gl, b):
        p, sb0 = group_psb(gl)
        pltpu.make_async_copy(
            tok_hbm.at[p, pl.ds(sb0 * 128, 256)], idx_v.at[b], sem_i.at[b]
        ).start()

    def idx_wait(b):
        pltpu.make_async_copy(
            tok_hbm.at[0, pl.ds(0, 256)], idx_v.at[b], sem_i.at[b]
        ).wait()

    def gather_start(b):
        for h in range(2):
            pltpu.make_async_copy(
                table_hbm.at[idx_v.at[b, pl.ds(128 * h, 128)]],
                rows_v.at[b, pl.ds(128 * h, 128)],
                sem_g.at[b],
            ).start()

    def gather_wait(b):
        for h in range(2):
            pltpu.make_async_copy(
                table_hbm.at[idx_v.at[b, pl.ds(0, 128)]],
                rows_v.at[b, pl.ds(128 * h, 128)],
                sem_g.at[b],
            ).wait()

    def out_start(gl, b):
        p, sb0 = group_psb(gl)
        for ss in range(2):
            for a in range(8):
                pltpu.make_async_copy(
                    yblk_v.at[b, ss, pl.ds(8 * a, 8)],
                    out_hbm.at[p, a, sb0 + ss],
                    sem_o.at[b],
                ).start()

    def out_wait(b):
        for _ in range(16):
            pltpu.make_async_copy(
                yblk_v.at[b, 0, pl.ds(0, 8)], out_hbm.at[0, 0, 0], sem_o.at[b]
            ).wait()

    idx_start(0, 0)
    idx_start(1, 1)
    idx_wait(0)
    gather_start(0)

    def step(go, _):
        for b in range(2):
            gl = go * 2 + b

            @pl.when(gl + 1 < G_PER_W)
            def _():
                idx_wait(1 - b)
                gather_start(1 - b)

            @pl.when(gl >= 2)
            def _():
                out_wait(b)

            gather_wait(b)

            @pl.when(gl + 2 < G_PER_W)
            def _():
                idx_start(gl + 2, b)

            rows = rows_v.at[b]

            for ss in range(2):
                yb = yblk_v.at[b, ss]

                @plsc.parallel_loop(0, D, step=1, unroll=4)
                def _(r):
                    ybr = yb.at[r]
                    col = jnp.broadcast_to(r, (L,))
                    for j in range(8):
                        vals = plsc.load_gather(
                            rows, [row_sel[8 * ss + j], col]
                        )
                        ybr[pl.ds(L * j, L)] = vals * SCALE

            out_start(gl, b)
        return 0

    lax.fori_loop(0, G_PER_W // 2, step, 0)
    out_wait(0)
    out_wait(1)


def kernel(tokens, embedding):
    tok_t = tokens.astype(jnp.int32).T
    emb_t = embedding.T
    mesh = plsc.VectorSubcoreMesh(core_axis_name="c", subcore_axis_name="s")

    tail = embedding[VOCAB - 64 :, :].reshape(32, 128)
    t2 = pl.kernel(
        _k1_body,
        out_type=jax.ShapeDtypeStruct((T2_ROWS, 128), jnp.float32),
        mesh=mesh,
        scratch_types=[
            pltpu.VMEM((2, D, 128), jnp.float32),
            pltpu.VMEM((2, D, 128), jnp.float32),
            pltpu.SemaphoreType.DMA((2,)),
            pltpu.SemaphoreType.DMA((2,)),
        ],
        compiler_params=pltpu.CompilerParams(
            use_tc_tiling_on_sc=True, needs_layout_passes=False
        ),
    )(emb_t, tail)

    table = t2.reshape(2 * T2_ROWS, D)

    out5 = pl.kernel(
        _k2_body,
        out_type=jax.ShapeDtypeStruct((P, 8, SB, 8, 128), jnp.float32),
        mesh=mesh,
        scratch_types=[
            pltpu.VMEM((2, 256), jnp.int32),
            pltpu.VMEM((2, 256, D), jnp.float32),
            pltpu.VMEM((2, 2, D, 128), jnp.float32),
            pltpu.SemaphoreType.DMA((2,)),
            pltpu.SemaphoreType.DMA((2,)),
            pltpu.SemaphoreType.DMA((2,)),
        ],
        compiler_params=pltpu.CompilerParams(
            use_tc_tiling_on_sc=False, needs_layout_passes=False
        ),
    )(tok_t, table)

    return out5.transpose(2, 4, 0, 1, 3).reshape(S, P, D)


# speedup vs baseline: 1.6191x; 1.5238x over previous
import math

import jax
import jax.numpy as jnp
from jax import lax
from jax.experimental import pallas as pl
from jax.experimental.pallas import tpu as pltpu
from jax.experimental.pallas import tpu_sc as plsc

VOCAB = 1000000
D = 64
SCALE = math.sqrt(D)
L = 16

NC = 2
NS = 16
NW = NC * NS

S = 4096
P = 200
SB = S // 128

TCOLS = (VOCAB + 127) // 128
K1_ITERS = (TCOLS + NW - 1) // NW
T2_ROWS = VOCAB // 2

GROUPS = (P * SB) // 2
G_PER_W = GROUPS // NW


def _k1_body(embt_hbm, tail_hbm, t2_hbm, buf_v, tbuf_v, sem_i, sem_o):
    wid = lax.axis_index("s") * NC + lax.axis_index("c")

    iota = lax.iota(jnp.int32, L)
    row_sel = [iota + L * k for k in range(D // L)]

    def col_start(i):
        return pl.multiple_of(jnp.minimum(i * NW + wid, TCOLS - 2) * 128, 128)

    def in_start(i, b):
        pltpu.make_async_copy(
            embt_hbm.at[:, pl.ds(col_start(i), 128)],
            buf_v.at[b, :, pl.ds(0, 128)],
            sem_i.at[b],
        ).start()

    def in_wait(i, b):
        pltpu.make_async_copy(
            embt_hbm.at[:, pl.ds(0, 128)], buf_v.at[b, :, pl.ds(0, 128)], sem_i.at[b]
        ).wait()

    def out_start(i, b):
        pltpu.make_async_copy(
            tbuf_v.at[b],
            t2_hbm.at[pl.ds(pl.multiple_of(col_start(i) // 2, 64), 64)],
            sem_o.at[b],
        ).start()

    def out_wait(b):
        pltpu.make_async_copy(
            tbuf_v.at[b], t2_hbm.at[pl.ds(0, 64)], sem_o.at[b]
        ).wait()

    in_start(0, 0)

    def step(i, _):
        for b in range(2):
            it = i * 2 + b

            @pl.when(it + 1 < K1_ITERS)
            def _():
                in_start(it + 1, 1 - b)

            @pl.when(it >= 2)
            def _():
                out_wait(b)

            in_wait(it, b)
            buf = buf_v.at[b]
            tbuf = tbuf_v.at[b]

            @plsc.parallel_loop(0, 64, step=1, unroll=4)
            def _(q):
                trow = tbuf.at[q]
                for par in range(2):
                    col = jnp.broadcast_to(2 * q + par, (L,))
                    for k in range(D // L):
                        vals = plsc.load_gather(buf, [row_sel[k], col])
                        trow[pl.ds(par * D + L * k, L)] = vals

            out_start(it, b)
        return 0

    lax.fori_loop(0, K1_ITERS // 2, step, 0)
    it = K1_ITERS - 1

    @pl.when(it >= 2)
    def _():
        out_wait(0)

    in_wait(it, 0)
    buf = buf_v.at[0]
    tbuf = tbuf_v.at[0]

    @plsc.parallel_loop(0, 64, step=1, unroll=4)
    def _(q):
        trow = tbuf.at[q]
        for par in range(2):
            col = jnp.broadcast_to(2 * q + par, (L,))
            for k in range(D // L):
                vals = plsc.load_gather(buf, [row_sel[k], col])
                trow[pl.ds(par * D + L * k, L)] = vals

    out_start(it, 0)
    out_wait(1)
    out_wait(0)

    @pl.when(wid == 0)
    def _():
        pltpu.sync_copy(tail_hbm, buf_v.at[0, pl.ds(0, 32), pl.ds(0, 128)])
        pltpu.sync_copy(buf_v.at[0, pl.ds(0, 32), pl.ds(0, 128)], t2_hbm.at[pl.ds(T2_ROWS - 32, 32)])


def _k2_body(tok_hbm, table_hbm, out_hbm, idx_v, rows_v, yblk_v, sem_i, sem_g, sem_o):
    wid = lax.axis_index("s") * NC + lax.axis_index("c")

    iota = lax.iota(jnp.int32, L)
    row_sel = [iota + L * j for j in range(256 // L)]

    def group_psb(gl):
        g = gl * NW + wid
        return g // 16, 2 * lax.rem(g, 16)

    def idx_start(gl, b):
        p, sb0 = group_psb(gl)
        pltpu.make_async_copy(
            tok_hbm.at[p, pl.ds(sb0 * 128, 256)], idx_v.at[b], sem_i.at[b]
        ).start()

    def idx_wait(b):
        pltpu.make_async_copy(
            tok_hbm.at[0, pl.ds(0, 256)], idx_v.at[b], sem_i.at[b]
        ).wait()

    def gather_start(b):
        for h in range(2):
            pltpu.make_async_copy(
                table_hbm.at[idx_v.at[b, pl.ds(128 * h, 128)]],
                rows_v.at[b, pl.ds(128 * h, 128)],
                sem_g.at[b],
            ).start()

    def gather_wait(b):
        for h in range(2):
            pltpu.make_async_copy(
                table_hbm.at[idx_v.at[b, pl.ds(0, 128)]],
                rows_v.at[b, pl.ds(128 * h, 128)],
                sem_g.at[b],
            ).wait()

    def out_start(gl, b):
        p, sb0 = group_psb(gl)
        for ss in range(2):
            for a in range(8):
                pltpu.make_async_copy(
                    yblk_v.at[b, ss, pl.ds(8 * a, 8), pl.ds(0, 128)],
                    out_hbm.at[p, a, sb0 + ss],
                    sem_o.at[b],
                ).start()

    def out_wait(b):
        for _ in range(16):
            pltpu.make_async_copy(
                yblk_v.at[b, 0, pl.ds(0, 8), pl.ds(0, 128)],
                out_hbm.at[0, 0, 0],
                sem_o.at[b],
            ).wait()

    idx_start(0, 0)
    idx_start(1, 1)
    idx_wait(0)
    gather_start(0)

    def step(go, _):
        for b in range(2):
            gl = go * 2 + b

            @pl.when(gl + 1 < G_PER_W)
            def _():
                idx_wait(1 - b)
                gather_start(1 - b)

            @pl.when(gl >= 2)
            def _():
                out_wait(b)

            gather_wait(b)

            @pl.when(gl + 2 < G_PER_W)
            def _():
                idx_start(gl + 2, b)

            rows = rows_v.at[b]

            for ss in range(2):
                yb = yblk_v.at[b, ss]

                @plsc.parallel_loop(0, 128, step=1, unroll=4)
                def _(i):
                    rrow = rows.at[128 * ss + i]
                    col = jnp.broadcast_to(i, (L,))
                    for k in range(D // L):
                        vals = rrow[pl.ds(L * k, L)] * SCALE
                        plsc.store_scatter(yb, [row_sel[k], col], vals)

            out_start(gl, b)
        return 0

    lax.fori_loop(0, G_PER_W // 2, step, 0)
    out_wait(0)
    out_wait(1)


def kernel(tokens, embedding):
    tok_t = tokens.astype(jnp.int32).T
    emb_t = embedding.T
    mesh = plsc.VectorSubcoreMesh(core_axis_name="c", subcore_axis_name="s")

    tail = embedding[VOCAB - 64 :, :].reshape(32, 128)
    t2 = pl.kernel(
        _k1_body,
        out_type=jax.ShapeDtypeStruct((T2_ROWS, 128), jnp.float32),
        mesh=mesh,
        scratch_types=[
            pltpu.VMEM((2, D, 129), jnp.float32),
            pltpu.VMEM((2, D, 128), jnp.float32),
            pltpu.SemaphoreType.DMA((2,)),
            pltpu.SemaphoreType.DMA((2,)),
        ],
        compiler_params=pltpu.CompilerParams(
            use_tc_tiling_on_sc=True, needs_layout_passes=False
        ),
    )(emb_t, tail)

    table = t2.reshape(2 * T2_ROWS, D)

    out5 = pl.kernel(
        _k2_body,
        out_type=jax.ShapeDtypeStruct((P, 8, SB, 8, 128), jnp.float32),
        mesh=mesh,
        scratch_types=[
            pltpu.VMEM((2, 256), jnp.int32),
            pltpu.VMEM((2, 256, D), jnp.float32),
            pltpu.VMEM((2, 2, D, 129), jnp.float32),
            pltpu.SemaphoreType.DMA((2,)),
            pltpu.SemaphoreType.DMA((2,)),
            pltpu.SemaphoreType.DMA((2,)),
        ],
        compiler_params=pltpu.CompilerParams(
            use_tc_tiling_on_sc=False, needs_layout_passes=False
        ),
    )(tok_t, table)

    return out5.transpose(2, 4, 0, 1, 3).reshape(S, P, D)
